# baseline (speedup 1.0000x reference)
import functools

import jax
import jax.numpy as jnp
from jax import lax
from jax.experimental import pallas as pl
from jax.experimental.pallas import tpu as pltpu
from jax.experimental.pallas import tpu_sc as plsc

K = 2
TILE = 256
BLK = 512
AUG = 128
NC, NS = 2, 16
NW = NC * NS
SCH = 64
CH = 32


def _route_body(x_ref, r_ref, xaug_ref, pos_ref, te_ref, counts_ref, ent_ref):
    t, d = x_ref.shape
    e_num = r_ref.shape[1]
    nt = te_ref.shape[1]
    nb = pos_ref.shape[0]

    xt = x_ref[...]
    lg = jax.lax.dot(xt, r_ref[...])

    iota_e = jax.lax.broadcasted_iota(jnp.int32, (t, e_num), 1)
    l0 = jnp.max(lg, axis=-1, keepdims=True)
    e0 = jnp.min(jnp.where(lg == l0, iota_e, e_num), axis=-1)
    lg1 = jnp.where(iota_e == e0[:, None], -jnp.inf, lg)
    l1 = jnp.max(lg1, axis=-1, keepdims=True)
    e1 = jnp.min(jnp.where(lg1 == l1, iota_e, e_num), axis=-1)

    z = jnp.exp(l1[:, 0] - l0[:, 0])
    w0 = 1.0 / (1.0 + z)
    w1 = 1.0 - w0

    onehot = ((e0[:, None] == iota_e).astype(jnp.float32)
              + (e1[:, None] == iota_e).astype(jnp.float32))
    counts = jnp.sum(onehot, axis=0)
    counts_ref[...] = counts[None, :]
    total = jnp.maximum(jnp.sum(counts), 1.0)
    loads = counts / total
    ent_ref[...] = (-jnp.sum(loads * jnp.log(loads + 1e-6))).reshape(1, 1)

    xaug_ref[:, :d] = xt
    aug = jnp.concatenate(
        [w0[:, None], w1[:, None],
         e0[:, None].astype(jnp.float32), e1[:, None].astype(jnp.float32),
         jnp.zeros((t, AUG - 4), jnp.float32)], axis=1)
    xaug_ref[:, d:] = aug

    pc = jnp.ceil(counts / TILE) * TILE
    tri_excl = (jax.lax.broadcasted_iota(jnp.int32, (e_num, e_num), 0)
                < jax.lax.broadcasted_iota(jnp.int32, (e_num, e_num), 1)
                ).astype(jnp.float32)
    start = jax.lax.dot(pc[None, :], tri_excl)

    ts_i = (start * (1.0 / TILE)).astype(jnp.int32)
    iota_nt = jax.lax.broadcasted_iota(jnp.int32, (1, nt), 1)
    te = jnp.zeros((1, nt), jnp.int32)
    for e in range(1, e_num):
        te = te + (iota_nt >= ts_i[:, e:e + 1]).astype(jnp.int32)
    te_ref[...] = te

    tri = (jax.lax.broadcasted_iota(jnp.int32, (BLK, BLK), 0)
           >= jax.lax.broadcasted_iota(jnp.int32, (BLK, BLK), 1)
           ).astype(jnp.float32)
    nbs = t // BLK
    ncol = nb * e_num
    cols = []
    for b in range(nb):
        sl = slice((b % nbs) * BLK, (b % nbs + 1) * BLK)
        eb = (e0 if b < nbs else e1)[sl]
        cols.append(eb[:, None])
    ebb = jnp.concatenate(cols, axis=1)
    eb_full = jnp.broadcast_to(
        ebb[:, :, None], (BLK, nb, e_num)).reshape(BLK, ncol)
    iota_c = jax.lax.broadcasted_iota(jnp.int32, (BLK, ncol), 1)
    oh_all = (eb_full == jnp.remainder(iota_c, e_num)).astype(jnp.float32)
    within = jax.lax.dot(tri, oh_all)
    bs = within[BLK - 1:BLK, :]
    r_i = jax.lax.broadcasted_iota(jnp.int32, (ncol, ncol), 0)
    c_i = jax.lax.broadcasted_iota(jnp.int32, (ncol, ncol), 1)
    cm = (((r_i // e_num) < (c_i // e_num))
          & (jnp.remainder(r_i, e_num) == jnp.remainder(c_i, e_num))
          ).astype(jnp.float32)
    carr = jax.lax.dot(bs, cm, precision=jax.lax.Precision.HIGHEST)
    start_full = jnp.broadcast_to(
        start.reshape(1, 1, e_num), (1, nb, e_num)).reshape(1, ncol)
    posm = oh_all * (within + carr + start_full - 1.0)
    for b in range(nb):
        posb = jnp.sum(posm[:, b * e_num:(b + 1) * e_num], axis=1)
        pos_ref[b, :] = posb.astype(jnp.int32)


def _mlp_body(te_ref, xs_ref, wug_ref, wd_ref, out_ref):
    i = pl.program_id(0)
    d = out_ref.shape[1]
    blk = xs_ref[...]
    xt = blk[:, :d]
    e_f = te_ref[i].astype(jnp.float32)
    w0c = blk[:, d:d + 1]
    w1c = blk[:, d + 1:d + 2]
    e0c = blk[:, d + 2:d + 3]
    e1c = blk[:, d + 3:d + 4]
    wrow = jnp.where(e0c == e_f, w0c, jnp.where(e1c == e_f, w1c, 0.0))
    ug = jax.lax.dot(xt, wug_ref[0])
    i_half = ug.shape[-1] // 2
    up = ug[:, :i_half]
    gate = ug[:, i_half:]
    h = up * (gate / (1.0 + jnp.exp(-gate)))
    y = jax.lax.dot(h, wd_ref[0])
    out_ref[...] = y * wrow


def kernel(x, router, w_up_gate, w_down):
    b, s, d = x.shape
    e_num = router.shape[1]
    i_dim = w_down.shape[1]
    t = b * s
    na = K * t
    nt = na // TILE + e_num
    npad = nt * TILE
    nb = na // BLK
    daug = d + AUG
    x_flat = x.reshape(t, d)

    xaug, pos, te, counts, ent = pl.pallas_call(
        _route_body,
        out_shape=[
            jax.ShapeDtypeStruct((t, daug), jnp.float32),
            jax.ShapeDtypeStruct((nb, BLK), jnp.int32),
            jax.ShapeDtypeStruct((1, nt), jnp.int32),
            jax.ShapeDtypeStruct((1, e_num), jnp.float32),
            jax.ShapeDtypeStruct((1, 1), jnp.float32),
        ],
    )(x_flat, router)
    pos_flat = pos.reshape(na)

    tpw2 = t // NW
    nch = tpw2 // SCH
    mesh = plsc.VectorSubcoreMesh(core_axis_name="c", subcore_axis_name="s")

    @functools.partial(
        pl.kernel, mesh=mesh,
        out_type=jax.ShapeDtypeStruct((npad, daug), jnp.float32),
        scratch_types=[
            pltpu.VMEM((SCH, daug), jnp.float32),
            pltpu.VMEM((SCH, daug), jnp.float32),
            pltpu.VMEM((SCH,), jnp.int32),
            pltpu.VMEM((SCH,), jnp.int32),
            pltpu.VMEM((SCH,), jnp.int32),
            pltpu.VMEM((SCH,), jnp.int32),
            pltpu.SemaphoreType.DMA,
            pltpu.SemaphoreType.DMA,
            pltpu.SemaphoreType.DMA,
            pltpu.SemaphoreType.DMA,
        ],
    )
    def _dispatch(xa_hbm, pos_hbm, xs_hbm,
                  rows0, rows1, i00, i01, i10, i11, ls0, ls1, ss0, ss1):
        cid = lax.axis_index("c")
        sid = lax.axis_index("s")
        wid = sid * NC + cid
        bufs = [(rows0, i00, i10), (rows1, i01, i11)]
        lsems = [ls0, ls1]
        ssems = [ss0, ss1]

        def loads(ch, bi):
            rv, iv0, iv1 = bufs[bi]
            tg = wid * tpw2 + ch * SCH
            sem = lsems[bi]
            return (pltpu.async_copy(xa_hbm.at[pl.ds(tg, SCH)], rv, sem),
                    pltpu.async_copy(pos_hbm.at[pl.ds(tg, SCH)], iv0, sem),
                    pltpu.async_copy(pos_hbm.at[pl.ds(t + tg, SCH)], iv1, sem))

        def scatters(bi):
            rv, iv0, iv1 = bufs[bi]
            sem = ssems[bi]
            return (pltpu.async_copy(rv, xs_hbm.at[iv0], sem),
                    pltpu.async_copy(rv, xs_hbm.at[iv1], sem))

        ld = {0: loads(0, 0)}
        sc = {}
        for ch in range(nch):
            bi = ch & 1
            for c in ld[bi]:
                c.wait()
            if ch + 1 < nch:
                ob = (ch + 1) & 1
                if ob in sc:
                    for s_ in sc.pop(ob):
                        s_.wait()
                ld[ob] = loads(ch + 1, ob)
            sc[bi] = scatters(bi)
        for bi in list(sc):
            for s_ in sc[bi]:
                s_.wait()

    xs = _dispatch(xaug, pos_flat)

    grid_spec = pltpu.PrefetchScalarGridSpec(
        num_scalar_prefetch=1,
        grid=(nt,),
        in_specs=[
            pl.BlockSpec((TILE, daug), lambda i, te: (i, 0)),
            pl.BlockSpec((1, d, 2 * i_dim), lambda i, te: (te[i], 0, 0)),
            pl.BlockSpec((1, i_dim, d), lambda i, te: (te[i], 0, 0)),
        ],
        out_specs=pl.BlockSpec((TILE, d), lambda i, te: (i, 0)),
    )
    ys = pl.pallas_call(
        _mlp_body,
        grid_spec=grid_spec,
        out_shape=jax.ShapeDtypeStruct((npad, d), jnp.float32),
        compiler_params=pltpu.CompilerParams(
            vmem_limit_bytes=100 * 1024 * 1024),
    )(te.reshape(nt), xs, w_up_gate, w_down)

    tpw = t // NW

    nch4 = tpw // CH

    @functools.partial(
        pl.kernel, mesh=mesh,
        out_type=jax.ShapeDtypeStruct((t, d), jnp.float32),
        scratch_types=[
            pltpu.VMEM((CH, d), jnp.float32),
            pltpu.VMEM((CH, d), jnp.float32),
            pltpu.VMEM((CH, d), jnp.float32),
            pltpu.VMEM((CH, d), jnp.float32),
            pltpu.VMEM((CH,), jnp.int32),
            pltpu.VMEM((CH,), jnp.int32),
            pltpu.VMEM((CH,), jnp.int32),
            pltpu.VMEM((CH,), jnp.int32),
            pltpu.SemaphoreType.DMA,
            pltpu.SemaphoreType.DMA,
            pltpu.SemaphoreType.DMA,
            pltpu.SemaphoreType.DMA,
        ],
    )
    def _combine(ys_hbm, pos_hbm, out_hbm,
                 g0a, g1a, g0b, g1b, i0a, i1a, i0b, i1b,
                 gsa, gsb, osa, osb):
        cid = lax.axis_index("c")
        sid = lax.axis_index("s")
        wid = sid * NC + cid
        sets = [(g0a, g1a, i0a, i1a, gsa, osa), (g0b, g1b, i0b, i1b, gsb, osb)]

        def start_gathers(ch, si):
            g0_v, g1_v, i0_v, i1_v, gsem, _ = sets[si]
            tg = wid * tpw + ch * CH
            pltpu.sync_copy(pos_hbm.at[pl.ds(tg, CH)], i0_v)
            pltpu.sync_copy(pos_hbm.at[pl.ds(t + tg, CH)], i1_v)
            return (pltpu.async_copy(ys_hbm.at[i0_v], g0_v, gsem),
                    pltpu.async_copy(ys_hbm.at[i1_v], g1_v, gsem))

        pend_store = [None, None]
        gth = {0: start_gathers(0, 0)}
        for ch in range(nch4):
            si = ch & 1
            if ch + 1 < nch4:
                os_ = (ch + 1) & 1
                if pend_store[os_] is not None:
                    pend_store[os_].wait()
                gth[ch + 1] = start_gathers(ch + 1, os_)
            for c in gth.pop(ch):
                c.wait()
            g0_v, g1_v, _, _, _, osem = sets[si]

            def row_body(r, carry):
                for j in range(d // 16):
                    sl = pl.ds(j * 16, 16)
                    g0_v[r, sl] = g0_v[r, sl] + g1_v[r, sl]
                return carry

            lax.fori_loop(0, CH, row_body, 0)
            tg = wid * tpw + ch * CH
            pend_store[si] = pltpu.async_copy(
                g0_v, out_hbm.at[pl.ds(tg, CH)], osem)
        for ps in pend_store:
            if ps is not None:
                ps.wait()

    out = _combine(ys, pos_flat)
    return out.reshape(b, s, d), counts[0], ent[0, 0]


# speedup vs baseline: 1.0115x; 1.0115x over previous
import functools

import jax
import jax.numpy as jnp
from jax import lax
from jax.experimental import pallas as pl
from jax.experimental.pallas import tpu as pltpu
from jax.experimental.pallas import tpu_sc as plsc

K = 2
TILE = 256
BLK = 512
AUG = 128
NC, NS = 2, 16
NW = NC * NS
SCH = 64
CH = 32


def _route_body(x_ref, r_ref, xaug_ref, pos_ref, te_ref, counts_ref, ent_ref):
    t, d = x_ref.shape
    e_num = r_ref.shape[1]
    nt = te_ref.shape[1]
    nb = pos_ref.shape[0]

    xt = x_ref[...]
    lg = jax.lax.dot(xt, r_ref[...])

    iota_e = jax.lax.broadcasted_iota(jnp.int32, (t, e_num), 1)
    l0 = jnp.max(lg, axis=-1, keepdims=True)
    e0 = jnp.min(jnp.where(lg == l0, iota_e, e_num), axis=-1)
    lg1 = jnp.where(iota_e == e0[:, None], -jnp.inf, lg)
    l1 = jnp.max(lg1, axis=-1, keepdims=True)
    e1 = jnp.min(jnp.where(lg1 == l1, iota_e, e_num), axis=-1)

    z = jnp.exp(l1[:, 0] - l0[:, 0])
    w0 = 1.0 / (1.0 + z)
    w1 = 1.0 - w0

    onehot = ((e0[:, None] == iota_e).astype(jnp.float32)
              + (e1[:, None] == iota_e).astype(jnp.float32))
    counts = jnp.sum(onehot, axis=0)
    counts_ref[...] = counts[None, :]
    total = jnp.maximum(jnp.sum(counts), 1.0)
    loads = counts / total
    ent_ref[...] = (-jnp.sum(loads * jnp.log(loads + 1e-6))).reshape(1, 1)

    xaug_ref[:, :d] = xt
    aug = jnp.concatenate(
        [w0[:, None], w1[:, None],
         e0[:, None].astype(jnp.float32), e1[:, None].astype(jnp.float32),
         jnp.zeros((t, AUG - 4), jnp.float32)], axis=1)
    xaug_ref[:, d:] = aug

    pc = jnp.ceil(counts / TILE) * TILE
    tri_excl = (jax.lax.broadcasted_iota(jnp.int32, (e_num, e_num), 0)
                < jax.lax.broadcasted_iota(jnp.int32, (e_num, e_num), 1)
                ).astype(jnp.float32)
    start = jax.lax.dot(pc[None, :], tri_excl)

    ts_i = (start * (1.0 / TILE)).astype(jnp.int32)
    iota_nt = jax.lax.broadcasted_iota(jnp.int32, (1, nt), 1)
    te = jnp.zeros((1, nt), jnp.int32)
    for e in range(1, e_num):
        te = te + (iota_nt >= ts_i[:, e:e + 1]).astype(jnp.int32)
    te_ref[...] = te

    tri = (jax.lax.broadcasted_iota(jnp.int32, (BLK, BLK), 0)
           >= jax.lax.broadcasted_iota(jnp.int32, (BLK, BLK), 1)
           ).astype(jnp.float32)
    nbs = t // BLK
    carry = jnp.zeros((1, e_num), jnp.float32)
    for b in range(nb):
        sl = slice((b % nbs) * BLK, (b % nbs + 1) * BLK)
        eb = (e0 if b < nbs else e1)[sl]
        iota_be = jax.lax.broadcasted_iota(jnp.int32, (BLK, e_num), 1)
        ohb = (eb[:, None] == iota_be).astype(jnp.float32)
        within = jax.lax.dot(tri, ohb)
        rank = (jnp.sum(within * ohb, axis=1) - 1.0
                + jnp.sum(ohb * carry, axis=1))
        posb = jnp.sum(ohb * start, axis=1) + rank
        pos_ref[b, :] = posb.astype(jnp.int32)
        carry = carry + jnp.sum(ohb, axis=0, keepdims=True)


def _mlp_body(te_ref, xs_ref, wug_ref, wd_ref, out_ref):
    i = pl.program_id(0)
    d = out_ref.shape[1]
    blk = xs_ref[...]
    xt = blk[:, :d]
    e_f = te_ref[i].astype(jnp.float32)
    w0c = blk[:, d:d + 1]
    w1c = blk[:, d + 1:d + 2]
    e0c = blk[:, d + 2:d + 3]
    e1c = blk[:, d + 3:d + 4]
    wrow = jnp.where(e0c == e_f, w0c, jnp.where(e1c == e_f, w1c, 0.0))
    ug = jax.lax.dot(xt, wug_ref[0])
    i_half = ug.shape[-1] // 2
    up = ug[:, :i_half]
    gate = ug[:, i_half:]
    h = up * (gate / (1.0 + jnp.exp(-gate)))
    y = jax.lax.dot(h, wd_ref[0])
    out_ref[...] = y * wrow


def kernel(x, router, w_up_gate, w_down):
    b, s, d = x.shape
    e_num = router.shape[1]
    i_dim = w_down.shape[1]
    t = b * s
    na = K * t
    nt = na // TILE + e_num
    npad = nt * TILE
    nb = na // BLK
    daug = d + AUG
    x_flat = x.reshape(t, d)

    xaug, pos, te, counts, ent = pl.pallas_call(
        _route_body,
        out_shape=[
            jax.ShapeDtypeStruct((t, daug), jnp.float32),
            jax.ShapeDtypeStruct((nb, BLK), jnp.int32),
            jax.ShapeDtypeStruct((1, nt), jnp.int32),
            jax.ShapeDtypeStruct((1, e_num), jnp.float32),
            jax.ShapeDtypeStruct((1, 1), jnp.float32),
        ],
    )(x_flat, router)
    pos_flat = pos.reshape(na)

    tpw2 = t // NW
    nch = tpw2 // SCH
    mesh = plsc.VectorSubcoreMesh(core_axis_name="c", subcore_axis_name="s")

    @functools.partial(
        pl.kernel, mesh=mesh,
        out_type=jax.ShapeDtypeStruct((npad, daug), jnp.float32),
        scratch_types=[
            pltpu.VMEM((SCH, daug), jnp.float32),
            pltpu.VMEM((SCH, daug), jnp.float32),
            pltpu.VMEM((SCH,), jnp.int32),
            pltpu.VMEM((SCH,), jnp.int32),
            pltpu.VMEM((SCH,), jnp.int32),
            pltpu.VMEM((SCH,), jnp.int32),
            pltpu.SemaphoreType.DMA,
            pltpu.SemaphoreType.DMA,
            pltpu.SemaphoreType.DMA,
            pltpu.SemaphoreType.DMA,
        ],
    )
    def _dispatch(xa_hbm, pos_hbm, xs_hbm,
                  rows0, rows1, i00, i01, i10, i11, ls0, ls1, ss0, ss1):
        cid = lax.axis_index("c")
        sid = lax.axis_index("s")
        wid = sid * NC + cid
        bufs = [(rows0, i00, i10), (rows1, i01, i11)]
        lsems = [ls0, ls1]
        ssems = [ss0, ss1]

        def loads(ch, bi):
            rv, iv0, iv1 = bufs[bi]
            tg = wid * tpw2 + ch * SCH
            sem = lsems[bi]
            return (pltpu.async_copy(xa_hbm.at[pl.ds(tg, SCH)], rv, sem),
                    pltpu.async_copy(pos_hbm.at[pl.ds(tg, SCH)], iv0, sem),
                    pltpu.async_copy(pos_hbm.at[pl.ds(t + tg, SCH)], iv1, sem))

        def scatters(bi):
            rv, iv0, iv1 = bufs[bi]
            sem = ssems[bi]
            return (pltpu.async_copy(rv, xs_hbm.at[iv0], sem),
                    pltpu.async_copy(rv, xs_hbm.at[iv1], sem))

        ld = {0: loads(0, 0)}
        sc = {}
        for ch in range(nch):
            bi = ch & 1
            for c in ld[bi]:
                c.wait()
            if ch + 1 < nch:
                ob = (ch + 1) & 1
                if ob in sc:
                    for s_ in sc.pop(ob):
                        s_.wait()
                ld[ob] = loads(ch + 1, ob)
            sc[bi] = scatters(bi)
        for bi in list(sc):
            for s_ in sc[bi]:
                s_.wait()

    xs = _dispatch(xaug, pos_flat)

    grid_spec = pltpu.PrefetchScalarGridSpec(
        num_scalar_prefetch=1,
        grid=(nt,),
        in_specs=[
            pl.BlockSpec((TILE, daug), lambda i, te: (i, 0)),
            pl.BlockSpec((1, d, 2 * i_dim), lambda i, te: (te[i], 0, 0)),
            pl.BlockSpec((1, i_dim, d), lambda i, te: (te[i], 0, 0)),
        ],
        out_specs=pl.BlockSpec((TILE, d), lambda i, te: (i, 0)),
    )
    ys = pl.pallas_call(
        _mlp_body,
        grid_spec=grid_spec,
        out_shape=jax.ShapeDtypeStruct((npad, d), jnp.float32),
        compiler_params=pltpu.CompilerParams(
            vmem_limit_bytes=100 * 1024 * 1024),
    )(te.reshape(nt), xs, w_up_gate, w_down)

    tpw = t // NW

    nch4 = tpw // CH

    @functools.partial(
        pl.kernel, mesh=mesh,
        out_type=jax.ShapeDtypeStruct((t, d), jnp.float32),
        scratch_types=[
            pltpu.VMEM((CH, d), jnp.float32),
            pltpu.VMEM((CH, d), jnp.float32),
            pltpu.VMEM((CH, d), jnp.float32),
            pltpu.VMEM((CH, d), jnp.float32),
            pltpu.VMEM((CH,), jnp.int32),
            pltpu.VMEM((CH,), jnp.int32),
            pltpu.VMEM((CH,), jnp.int32),
            pltpu.VMEM((CH,), jnp.int32),
            pltpu.SemaphoreType.DMA,
            pltpu.SemaphoreType.DMA,
            pltpu.SemaphoreType.DMA,
            pltpu.SemaphoreType.DMA,
        ],
    )
    def _combine(ys_hbm, pos_hbm, out_hbm,
                 g0a, g1a, g0b, g1b, i0a, i1a, i0b, i1b,
                 gsa, gsb, osa, osb):
        cid = lax.axis_index("c")
        sid = lax.axis_index("s")
        wid = sid * NC + cid
        sets = [(g0a, g1a, i0a, i1a, gsa, osa), (g0b, g1b, i0b, i1b, gsb, osb)]

        def start_gathers(ch, si):
            g0_v, g1_v, i0_v, i1_v, gsem, _ = sets[si]
            tg = wid * tpw + ch * CH
            pltpu.sync_copy(pos_hbm.at[pl.ds(tg, CH)], i0_v)
            pltpu.sync_copy(pos_hbm.at[pl.ds(t + tg, CH)], i1_v)
            return (pltpu.async_copy(ys_hbm.at[i0_v], g0_v, gsem),
                    pltpu.async_copy(ys_hbm.at[i1_v], g1_v, gsem))

        pend_store = [None, None]
        gth = {0: start_gathers(0, 0)}
        for ch in range(nch4):
            si = ch & 1
            if ch + 1 < nch4:
                os_ = (ch + 1) & 1
                if pend_store[os_] is not None:
                    pend_store[os_].wait()
                gth[ch + 1] = start_gathers(ch + 1, os_)
            for c in gth.pop(ch):
                c.wait()
            g0_v, g1_v, _, _, _, osem = sets[si]

            def row_body(r, carry):
                for j in range(d // 16):
                    sl = pl.ds(j * 16, 16)
                    g0_v[r, sl] = g0_v[r, sl] + g1_v[r, sl]
                return carry

            lax.fori_loop(0, CH, row_body, 0)
            tg = wid * tpw + ch * CH
            pend_store[si] = pltpu.async_copy(
                g0_v, out_hbm.at[pl.ds(tg, CH)], osem)
        for ps in pend_store:
            if ps is not None:
                ps.wait()

    out = _combine(ys, pos_flat)
    return out.reshape(b, s, d), counts[0], ent[0, 0]
